# Initial kernel scaffold; baseline (speedup 1.0000x reference)
import jax
import jax.numpy as jnp
from jax.experimental import pallas as pl


def kernel(node_idx, time_feat, dwell_feat, edge_index, batch, embed, W_in, b_in, ggnn_w, gru_w_ih, gru_w_hh, gru_b_ih, gru_b_hh, attn_in_w, attn_in_b, attn_out_w, attn_out_b, lin_w, lin_b):
    raise NotImplementedError("write your pallas kernel here")


# speedup vs baseline: 1.6204x; 1.6204x over previous
import functools
import math

import jax
import jax.numpy as jnp
from jax import lax
from jax.experimental import pallas as pl
from jax.experimental.pallas import tpu as pltpu
from jax.experimental.pallas import tpu_sc as plsc

_N = 50000
_NP = 50176
_PER_TILE = 1568
_GCHUNK = 112
_E = 800000
_EP = 800768
_ET = 50048
_ESTEPS = 391
_C = 12544
_CROWS = _C + 16
_PT_ROWS = _C // 16
_B = 500
_H = 128

def _sc_gather_body(table_hbm, idx_hbm, out_hbm, idx_v, rows_v, sem):
    wid = lax.axis_index("s") * 2 + lax.axis_index("c")
    base = wid * _PER_TILE

    def step(j, carry):
        off = base + j * _GCHUNK
        pltpu.sync_copy(idx_hbm.at[pl.ds(off, _GCHUNK)], idx_v)
        pltpu.async_copy(table_hbm.at[idx_v], rows_v, sem).wait()
        pltpu.sync_copy(rows_v, out_hbm.at[pl.ds(off, _GCHUNK)])
        return carry

    lax.fori_loop(0, _PER_TILE // _GCHUNK, step, 0)


@functools.cache
def _get_sc_gather():
    return pl.kernel(
        _sc_gather_body,
        out_type=jax.ShapeDtypeStruct((_NP, _H), jnp.float32),
        mesh=plsc.VectorSubcoreMesh(core_axis_name="c", subcore_axis_name="s"),
        scratch_types=[
            pltpu.VMEM((_GCHUNK,), jnp.int32),
            pltpu.VMEM((_GCHUNK, _H), jnp.float32),
            pltpu.SemaphoreType.DMA,
        ],
    )


def _sc_edge_agg_body(m_hbm, src_hbm, dst_hbm, zeros_hbm, out_hbm,
                      sidx_v, draw_v, didx_v, rows_v, acc, sem):
    cid = lax.axis_index("c")
    tid = lax.axis_index("s")

    for k in range(2):
        cbase = (2 * cid + k) * _C
        pltpu.sync_copy(zeros_hbm, acc.at[pl.ds(tid * _PT_ROWS, _PT_ROWS)])

        @pl.when(tid == 0)
        def _zero_garbage():
            pltpu.sync_copy(zeros_hbm.at[pl.ds(0, 16)], acc.at[pl.ds(_C, 16)])

        plsc.subcore_barrier()

        def step(s, carry):
            ebase = tid * _ET + s * 128
            pltpu.sync_copy(src_hbm.at[pl.ds(ebase, 128)], sidx_v)
            pltpu.sync_copy(dst_hbm.at[pl.ds(ebase, 128)], draw_v)
            for j in range(8):
                d = draw_v[pl.ds(j * 16, 16)]
                loc = d - cbase
                oob = jnp.logical_or(loc < 0, loc >= _C)
                didx_v[pl.ds(j * 16, 16)] = jnp.where(
                    oob, jnp.full((16,), _C, jnp.int32), loc)
            pltpu.async_copy(m_hbm.at[sidx_v], rows_v, sem).wait()
            pltpu.sync_copy(rows_v, acc.at[didx_v], add=True)
            return carry

        lax.fori_loop(0, _ESTEPS, step, 0)
        plsc.subcore_barrier()

        def copy_out(i, carry):
            r0 = tid * _PT_ROWS + i * _GCHUNK
            pltpu.sync_copy(acc.at[pl.ds(r0, _GCHUNK)],
                            out_hbm.at[pl.ds(cbase + r0, _GCHUNK)])
            return carry

        lax.fori_loop(0, _PT_ROWS // _GCHUNK, copy_out, 0)
        plsc.subcore_barrier()


@functools.cache
def _get_sc_edge_agg():
    return pl.kernel(
        _sc_edge_agg_body,
        out_type=jax.ShapeDtypeStruct((_NP, _H), jnp.float32),
        mesh=plsc.VectorSubcoreMesh(core_axis_name="c", subcore_axis_name="s"),
        scratch_types=[
            pltpu.VMEM((128,), jnp.int32),
            pltpu.VMEM((128,), jnp.int32),
            pltpu.VMEM((128,), jnp.int32),
            pltpu.VMEM((128, _H), jnp.float32),
            pltpu.VMEM_SHARED((_CROWS, _H), jnp.float32),
            pltpu.SemaphoreType.DMA,
        ],
    )


_R = 512
_G = _NP // _R


def _tc_in_body(g_ref, f_ref, winT_ref, wf8_ref, b_ref, w0_ref, x_ref, m_ref):
    x = jnp.maximum(
        g_ref[...] @ winT_ref[...] + f_ref[...] @ wf8_ref[...] + b_ref[...],
        0.0)
    x_ref[...] = x
    m_ref[...] = x @ w0_ref[...]


_tc_in = pl.pallas_call(
    _tc_in_body,
    grid=(_G,),
    in_specs=[
        pl.BlockSpec((_R, _H), lambda i: (i, 0)),
        pl.BlockSpec((_R, 8), lambda i: (i, 0)),
        pl.BlockSpec((_H, _H), lambda i: (0, 0)),
        pl.BlockSpec((8, _H), lambda i: (0, 0)),
        pl.BlockSpec((1, _H), lambda i: (0, 0)),
        pl.BlockSpec((_H, _H), lambda i: (0, 0)),
    ],
    out_specs=[
        pl.BlockSpec((_R, _H), lambda i: (i, 0)),
        pl.BlockSpec((_R, _H), lambda i: (i, 0)),
    ],
    out_shape=[
        jax.ShapeDtypeStruct((_NP, _H), jnp.float32),
        jax.ShapeDtypeStruct((_NP, _H), jnp.float32),
    ],
)


def _tc_gru_body(x_ref, agg_ref, wihT_ref, whhT_ref, bih_ref, bhh_ref,
                 wn_ref, bn_ref, xo_ref, mo_ref):
    x = x_ref[...]
    gi = agg_ref[...] @ wihT_ref[...] + bih_ref[...]
    gh = x @ whhT_ref[...] + bhh_ref[...]
    r = jax.nn.sigmoid(gi[:, :_H] + gh[:, :_H])
    z = jax.nn.sigmoid(gi[:, _H:2 * _H] + gh[:, _H:2 * _H])
    n = jnp.tanh(gi[:, 2 * _H:] + r * gh[:, 2 * _H:])
    xn = (1.0 - z) * n + z * x
    xo_ref[...] = xn
    mo_ref[...] = xn @ wn_ref[...] + bn_ref[...]


def _mk_tc_gru(K):
    return pl.pallas_call(
        _tc_gru_body,
        grid=(_G,),
        in_specs=[
            pl.BlockSpec((_R, _H), lambda i: (i, 0)),
            pl.BlockSpec((_R, _H), lambda i: (i, 0)),
            pl.BlockSpec((_H, 3 * _H), lambda i: (0, 0)),
            pl.BlockSpec((_H, 3 * _H), lambda i: (0, 0)),
            pl.BlockSpec((1, 3 * _H), lambda i: (0, 0)),
            pl.BlockSpec((1, 3 * _H), lambda i: (0, 0)),
            pl.BlockSpec((_H, K), lambda i: (0, 0)),
            pl.BlockSpec((1, K), lambda i: (0, 0)),
        ],
        out_specs=[
            pl.BlockSpec((_R, _H), lambda i: (i, 0)),
            pl.BlockSpec((_R, K), lambda i: (i, 0)),
        ],
        out_shape=[
            jax.ShapeDtypeStruct((_NP, _H), jnp.float32),
            jax.ShapeDtypeStruct((_NP, K), jnp.float32),
        ],
    )


_tc_gru_128 = _mk_tc_gru(_H)
_tc_gru_384 = _mk_tc_gru(3 * _H)


def _tc_qlast_body(qkv_ref, batch_ref, il_ref, out_ref):
    @pl.when(pl.program_id(0) == 0)
    def _init():
        out_ref[...] = jnp.zeros_like(out_ref)

    b = batch_ref[...]
    E = (b == lax.broadcasted_iota(jnp.int32, (_R, _B), 1)).astype(jnp.float32)
    Em = E * il_ref[...]
    q = qkv_ref[:, :_H]
    out_ref[...] += lax.dot_general(Em, q, (((0,), (0,)), ((), ())))


_tc_qlast = pl.pallas_call(
    _tc_qlast_body,
    grid=(_G,),
    in_specs=[
        pl.BlockSpec((_R, 3 * _H), lambda i: (i, 0)),
        pl.BlockSpec((_R, 1), lambda i: (i, 0)),
        pl.BlockSpec((_R, 1), lambda i: (i, 0)),
    ],
    out_specs=pl.BlockSpec((_B, _H), lambda i: (0, 0)),
    out_shape=jax.ShapeDtypeStruct((_B, _H), jnp.float32),
)


def _tc_attn_body(qkv_ref, batch_ref, qlast_ref, S_ref, ST_ref,
                  num_ref, den_ref):
    @pl.when(pl.program_id(0) == 0)
    def _init():
        num_ref[...] = jnp.zeros_like(num_ref)
        den_ref[...] = jnp.zeros_like(den_ref)

    b = batch_ref[...]
    E = (b == lax.broadcasted_iota(jnp.int32, (_R, _B), 1)).astype(jnp.float32)
    qb = E @ qlast_ref[...]
    k = qkv_ref[:, _H:2 * _H]
    v = qkv_ref[:, 2 * _H:]
    s4 = ((qb * k) @ S_ref[...]) * (1.0 / math.sqrt(32.0))
    w = jnp.exp(s4)
    wf = w @ ST_ref[...]
    num_ref[...] += lax.dot_general(E, wf * v, (((0,), (0,)), ((), ())))
    den_ref[...] += lax.dot_general(E, w, (((0,), (0,)), ((), ())))


_tc_attn = pl.pallas_call(
    _tc_attn_body,
    grid=(_G,),
    in_specs=[
        pl.BlockSpec((_R, 3 * _H), lambda i: (i, 0)),
        pl.BlockSpec((_R, 1), lambda i: (i, 0)),
        pl.BlockSpec((_B, _H), lambda i: (0, 0)),
        pl.BlockSpec((_H, 4), lambda i: (0, 0)),
        pl.BlockSpec((4, _H), lambda i: (0, 0)),
    ],
    out_specs=[
        pl.BlockSpec((_B, _H), lambda i: (0, 0)),
        pl.BlockSpec((_B, 4), lambda i: (0, 0)),
    ],
    out_shape=[
        jax.ShapeDtypeStruct((_B, _H), jnp.float32),
        jax.ShapeDtypeStruct((_B, 4), jnp.float32),
    ],
)

_CB = 2048


def _tc_head_body(num_ref, den_ref, ST_ref, woT_ref, bo_ref,
                  linT_ref, lb_ref, out_ref):
    denf = den_ref[...] @ ST_ref[...]
    ctx = num_ref[...] / denf
    fin = ctx @ woT_ref[...] + bo_ref[...]
    out_ref[...] = fin @ linT_ref[...] + lb_ref[...]


_tc_head = pl.pallas_call(
    _tc_head_body,
    grid=(pl.cdiv(10000, _CB),),
    in_specs=[
        pl.BlockSpec((_B, _H), lambda j: (0, 0)),
        pl.BlockSpec((_B, 4), lambda j: (0, 0)),
        pl.BlockSpec((4, _H), lambda j: (0, 0)),
        pl.BlockSpec((_H, _H), lambda j: (0, 0)),
        pl.BlockSpec((1, _H), lambda j: (0, 0)),
        pl.BlockSpec((_H, _CB), lambda j: (0, j)),
        pl.BlockSpec((1, _CB), lambda j: (0, j)),
    ],
    out_specs=pl.BlockSpec((_B, _CB), lambda j: (0, j)),
    out_shape=jax.ShapeDtypeStruct((_B, 10000), jnp.float32),
)


def kernel(node_idx, time_feat, dwell_feat, edge_index, batch, embed,
           W_in, b_in, ggnn_w, gru_w_ih, gru_w_hh, gru_b_ih, gru_b_hh,
           attn_in_w, attn_in_b, attn_out_w, attn_out_b, lin_w, lin_b):
    f32 = jnp.float32
    table = jnp.pad(embed, ((0, 0), (0, 2)))
    idx_pad = jnp.pad(node_idx.astype(jnp.int32), (0, _NP - _N))
    src = jnp.pad(edge_index[0].astype(jnp.int32), (0, _EP - _E))
    dst = jnp.pad(edge_index[1].astype(jnp.int32), (0, _EP - _E),
                  constant_values=1 << 28)
    b32 = batch.astype(jnp.int32)
    batch2d = jnp.pad(b32, (0, _NP - _N), constant_values=_B)[:, None]
    il = jnp.concatenate([b32[1:] != b32[:-1],
                          jnp.ones((1,), bool)])
    il2d = jnp.pad(il.astype(f32), (0, _NP - _N))[:, None]
    f_pad = (jnp.zeros((_NP, 8), f32)
             .at[:_N, 0].set(time_feat[:, 0])
             .at[:_N, 1].set(dwell_feat[:, 0]))
    wf8 = (jnp.zeros((8, _H), f32)
           .at[0].set(W_in[:, 126])
           .at[1].set(W_in[:, 127]))
    zeros_pt = jnp.zeros((_PT_ROWS, _H), f32)
    S = jnp.repeat(jnp.eye(4, dtype=f32), 32, axis=0)

    g = _get_sc_gather()(table, idx_pad)
    x, m = _tc_in(g, f_pad, W_in.T, wf8, b_in[None, :], ggnn_w[0])
    agg = _get_sc_edge_agg()(m, src, dst, zeros_pt)
    x, m = _tc_gru_128(x, agg, gru_w_ih.T, gru_w_hh.T, gru_b_ih[None, :],
                       gru_b_hh[None, :], ggnn_w[1], jnp.zeros((1, _H), f32))
    agg = _get_sc_edge_agg()(m, src, dst, zeros_pt)
    x, qkv = _tc_gru_384(x, agg, gru_w_ih.T, gru_w_hh.T, gru_b_ih[None, :],
                         gru_b_hh[None, :], attn_in_w.T, attn_in_b[None, :])
    qlast = _tc_qlast(qkv, batch2d, il2d)
    num, den = _tc_attn(qkv, batch2d, qlast, S, S.T)
    return _tc_head(num, den, S.T, attn_out_w.T, attn_out_b[None, :],
                    lin_w.T, lin_b[None, :])
